# baseline (speedup 1.0000x reference)
import functools

import jax
import jax.numpy as jnp
from jax import lax
from jax.experimental import pallas as pl
from jax.experimental.pallas import tpu as pltpu
from jax.experimental.pallas import tpu_sc as plsc

CH = 128
NBUF = 5
SKEW = 3


def _gather_body(feat_hbm, idx_hbm, out_hbm, *refs, m, per_w, nc):
    idx_all = refs[0]
    rows_v = refs[1:1 + NBUF]
    isem = refs[1 + NBUF]
    gsem = refs[2 + NBUF:2 + 2 * NBUF]
    ssem = refs[2 + 2 * NBUF:2 + 3 * NBUF]

    wid = lax.axis_index("s") * nc + lax.axis_index("c")
    base = jnp.minimum(wid * per_w, m - per_w)
    nch = per_w // CH

    def idx_slice(k):
        return idx_all.at[pl.ds(k * CH, CH)]

    def start_gather(b, k):
        pltpu.async_copy(feat_hbm.at[idx_slice(k)], rows_v[b], gsem[b])

    def wait_gather(b, k):
        pltpu.make_async_copy(
            feat_hbm.at[idx_slice(k)], rows_v[b], gsem[b]
        ).wait()

    def wait_store(b, k):
        pltpu.make_async_copy(
            rows_v[b], out_hbm.at[pl.ds(base + k * CH, CH), :], ssem[b]
        ).wait()

    pltpu.async_copy(idx_hbm.at[pl.ds(base, per_w)], idx_all, isem).wait()
    for j in range(SKEW):
        start_gather(j, j)

    @pl.loop(0, nch, step=NBUF)
    def _block(c):
        for b in range(NBUF):
            k = c + b
            bs = (b + SKEW) % NBUF

            @pl.when(k + SKEW < nch)
            def _():
                @pl.when(k + SKEW >= NBUF)
                def _():
                    wait_store(bs, k + SKEW - NBUF)
                start_gather(bs, k + SKEW)

            wait_gather(b, k)
            pltpu.async_copy(
                rows_v[b], out_hbm.at[pl.ds(base + k * CH, CH), :], ssem[b]
            )

    for b in range(NBUF):
        wait_store(b, 0)


def kernel(features, indices):
    m = indices.shape[1]
    d = features.shape[1]
    idx = indices.reshape(m)
    if idx.dtype != jnp.int32:
        idx = idx.astype(jnp.int32)
    info = plsc.get_sparse_core_info()
    nc, ns = info.num_cores, info.num_subcores
    nw = nc * ns
    per_w_rows = -(-m // nw)
    chunks = -(-per_w_rows // CH)
    chunks = -(-chunks // NBUF) * NBUF
    per_w = chunks * CH
    mesh = plsc.VectorSubcoreMesh(core_axis_name="c", subcore_axis_name="s")
    scratch = (
        [pltpu.VMEM((per_w,), jnp.int32)]
        + [pltpu.VMEM((CH, d), jnp.float32) for _ in range(NBUF)]
        + [pltpu.SemaphoreType.DMA for _ in range(1 + 2 * NBUF)]
    )
    k = pl.kernel(
        functools.partial(_gather_body, m=m, per_w=per_w, nc=nc),
        out_type=jax.ShapeDtypeStruct((m, d), features.dtype),
        mesh=mesh,
        scratch_types=scratch,
    )
    return k(features, idx)


# speedup vs baseline: 1.0206x; 1.0206x over previous
import functools

import jax
import jax.numpy as jnp
from jax import lax
from jax.experimental import pallas as pl
from jax.experimental.pallas import tpu as pltpu
from jax.experimental.pallas import tpu_sc as plsc

CH = 128
NBUF = 6
SKEW = 3


def _gather_body(feat_hbm, idx_hbm, out_hbm, *refs, m, per_w, tail, nc):
    idx_all = refs[0]
    rows_v = refs[1:1 + NBUF]
    tail_v = refs[1 + NBUF]
    isem = refs[2 + NBUF]
    tsem = refs[3 + NBUF]
    gsem = refs[4 + NBUF:4 + 2 * NBUF]
    ssem = refs[4 + 2 * NBUF:4 + 3 * NBUF]

    wid = lax.axis_index("s") * nc + lax.axis_index("c")
    base = jnp.minimum(wid * per_w, m - per_w)
    nfull = (per_w - tail) // CH

    def start_gather(b, k):
        pltpu.async_copy(
            feat_hbm.at[idx_all.at[pl.ds(k * CH, CH)]], rows_v[b], gsem[b]
        )

    def wait_gather(b, k):
        pltpu.make_async_copy(
            feat_hbm.at[idx_all.at[pl.ds(k * CH, CH)]], rows_v[b], gsem[b]
        ).wait()

    def wait_store(b, k):
        pltpu.make_async_copy(
            rows_v[b], out_hbm.at[pl.ds(base + k * CH, CH), :], ssem[b]
        ).wait()

    pltpu.async_copy(idx_hbm.at[pl.ds(base, per_w)], idx_all, isem).wait()
    if tail:
        pltpu.async_copy(
            feat_hbm.at[idx_all.at[pl.ds(nfull * CH, tail)]], tail_v, tsem
        )
    for j in range(SKEW):
        start_gather(j, j)

    @pl.loop(0, nfull, step=NBUF)
    def _block(c):
        for b in range(NBUF):
            k = c + b
            bs = (b + SKEW) % NBUF

            @pl.when(k + SKEW < nfull)
            def _():
                @pl.when(k + SKEW >= NBUF)
                def _():
                    wait_store(bs, k + SKEW - NBUF)
                start_gather(bs, k + SKEW)

            wait_gather(b, k)
            pltpu.async_copy(
                rows_v[b], out_hbm.at[pl.ds(base + k * CH, CH), :], ssem[b]
            )

    if tail:
        pltpu.make_async_copy(
            feat_hbm.at[idx_all.at[pl.ds(nfull * CH, tail)]], tail_v, tsem
        ).wait()
        pltpu.async_copy(
            tail_v, out_hbm.at[pl.ds(base + nfull * CH, tail), :], tsem
        )

    for b in range(NBUF):
        wait_store(b, 0)
    if tail:
        pltpu.make_async_copy(
            tail_v, out_hbm.at[pl.ds(base, tail), :], tsem
        ).wait()


def kernel(features, indices):
    m = indices.shape[1]
    d = features.shape[1]
    idx = indices.reshape(m)
    if idx.dtype != jnp.int32:
        idx = idx.astype(jnp.int32)
    info = plsc.get_sparse_core_info()
    nc, ns = info.num_cores, info.num_subcores
    nw = nc * ns
    per_w = -(-(-(-m // nw)) // 8) * 8
    nfull = per_w // CH
    while nfull % NBUF:
        nfull -= 1
    tail = per_w - nfull * CH
    mesh = plsc.VectorSubcoreMesh(core_axis_name="c", subcore_axis_name="s")
    scratch = (
        [pltpu.VMEM((per_w,), jnp.int32)]
        + [pltpu.VMEM((CH, d), jnp.float32) for _ in range(NBUF)]
        + [pltpu.VMEM((max(tail, 8), d), jnp.float32)]
        + [pltpu.SemaphoreType.DMA for _ in range(2 + 2 * NBUF)]
    )
    k = pl.kernel(
        functools.partial(_gather_body, m=m, per_w=per_w, tail=tail, nc=nc),
        out_type=jax.ShapeDtypeStruct((m, d), features.dtype),
        mesh=mesh,
        scratch_types=scratch,
    )
    return k(features, idx)


# speedup vs baseline: 1.0262x; 1.0055x over previous
import functools

import jax
import jax.numpy as jnp
from jax import lax
from jax.experimental import pallas as pl
from jax.experimental.pallas import tpu as pltpu
from jax.experimental.pallas import tpu_sc as plsc

CH = 128
NBUF = 6
SKEW = 4


def _gather_body(feat_hbm, idx_hbm, out_hbm, *refs, m, per_w, tail, nc):
    idx_all = refs[0]
    rows_v = refs[1:1 + NBUF]
    tail_v = refs[1 + NBUF]
    isem = refs[2 + NBUF]
    tsem = refs[3 + NBUF]
    gsem = refs[4 + NBUF:4 + 2 * NBUF]
    ssem = refs[4 + 2 * NBUF:4 + 3 * NBUF]

    wid = lax.axis_index("s") * nc + lax.axis_index("c")
    base = jnp.minimum(wid * per_w, m - per_w)
    nfull = (per_w - tail) // CH

    def start_gather(b, k):
        pltpu.async_copy(
            feat_hbm.at[idx_all.at[pl.ds(k * CH, CH)]], rows_v[b], gsem[b]
        )

    def wait_gather(b, k):
        pltpu.make_async_copy(
            feat_hbm.at[idx_all.at[pl.ds(k * CH, CH)]], rows_v[b], gsem[b]
        ).wait()

    def wait_store(b, k):
        pltpu.make_async_copy(
            rows_v[b], out_hbm.at[pl.ds(base + k * CH, CH), :], ssem[b]
        ).wait()

    pltpu.async_copy(idx_hbm.at[pl.ds(base, per_w)], idx_all, isem).wait()
    if tail:
        pltpu.async_copy(
            feat_hbm.at[idx_all.at[pl.ds(nfull * CH, tail)]], tail_v, tsem
        )
    for j in range(SKEW):
        start_gather(j, j)

    @pl.loop(0, nfull, step=NBUF)
    def _block(c):
        for b in range(NBUF):
            k = c + b
            bs = (b + SKEW) % NBUF

            @pl.when(k + SKEW < nfull)
            def _():
                @pl.when(k + SKEW >= NBUF)
                def _():
                    wait_store(bs, k + SKEW - NBUF)
                start_gather(bs, k + SKEW)

            wait_gather(b, k)
            pltpu.async_copy(
                rows_v[b], out_hbm.at[pl.ds(base + k * CH, CH), :], ssem[b]
            )

    if tail:
        pltpu.make_async_copy(
            feat_hbm.at[idx_all.at[pl.ds(nfull * CH, tail)]], tail_v, tsem
        ).wait()
        pltpu.async_copy(
            tail_v, out_hbm.at[pl.ds(base + nfull * CH, tail), :], tsem
        )

    for b in range(NBUF):
        wait_store(b, 0)
    if tail:
        pltpu.make_async_copy(
            tail_v, out_hbm.at[pl.ds(base, tail), :], tsem
        ).wait()


def kernel(features, indices):
    m = indices.shape[1]
    d = features.shape[1]
    idx = indices.reshape(m)
    if idx.dtype != jnp.int32:
        idx = idx.astype(jnp.int32)
    info = plsc.get_sparse_core_info()
    nc, ns = info.num_cores, info.num_subcores
    nw = nc * ns
    per_w = -(-(-(-m // nw)) // 8) * 8
    nfull = per_w // CH
    while nfull % NBUF:
        nfull -= 1
    tail = per_w - nfull * CH
    mesh = plsc.VectorSubcoreMesh(core_axis_name="c", subcore_axis_name="s")
    scratch = (
        [pltpu.VMEM((per_w,), jnp.int32)]
        + [pltpu.VMEM((CH, d), jnp.float32) for _ in range(NBUF)]
        + [pltpu.VMEM((max(tail, 8), d), jnp.float32)]
        + [pltpu.SemaphoreType.DMA for _ in range(2 + 2 * NBUF)]
    )
    k = pl.kernel(
        functools.partial(_gather_body, m=m, per_w=per_w, tail=tail, nc=nc),
        out_type=jax.ShapeDtypeStruct((m, d), features.dtype),
        mesh=mesh,
        scratch_types=scratch,
    )
    return k(features, idx)
